# baseline (speedup 1.0000x reference)
import functools

import jax
import jax.numpy as jnp
from jax import lax
from jax.experimental import pallas as pl
from jax.experimental.pallas import tpu as pltpu
from jax.experimental.pallas import tpu_sc as plsc

_NC = 2
_NS = 16
_NW = _NC * _NS

_CHUNK = 1024


_SUPER = 10


def _embed_body(idx_hbm, table_hbm, out_hbm,
                idx_super, rows0, rows1, gsem, wsem):
    D = table_hbm.shape[1]
    V = table_hbm.shape[0]
    B = idx_hbm.shape[0]
    per_w = B // _NW
    chunks_per_w = per_w // _CHUNK
    n_super = chunks_per_w // _SUPER

    wid = lax.axis_index("s") * _NC + lax.axis_index("c")
    base = wid * per_w

    rowsb = (rows0, rows1)

    def super_chunk(s, carry):
        sbase = base + s * _SUPER * _CHUNK
        pltpu.sync_copy(idx_hbm.at[pl.ds(sbase, _SUPER * _CHUNK)], idx_super)

        def wb_start(c):
            return pltpu.async_copy(
                rowsb[c % 2], out_hbm.at[pl.ds(sbase + c * _CHUNK, _CHUNK)],
                wsem)

        wh = None
        for c in range(_SUPER):
            p = c % 2
            pltpu.async_copy(
                table_hbm.at[idx_super.at[pl.ds(c * _CHUNK, _CHUNK)]],
                rowsb[p], gsem).wait()
            if wh is not None:
                wh.wait()
            wh = wb_start(c)
        wh.wait()
        return carry

    lax.fori_loop(0, n_super, super_chunk, 0)


def kernel(indices, table):
    R, C = indices.shape
    V, D = table.shape
    B = R * C
    assert B % (_NW * _CHUNK * _SUPER) == 0

    idx_flat = indices.reshape(B)

    mesh = plsc.VectorSubcoreMesh(core_axis_name="c", subcore_axis_name="s")
    embed = functools.partial(
        pl.kernel,
        out_type=jax.ShapeDtypeStruct((B, D), jnp.float32),
        mesh=mesh,
        scratch_types=[
            pltpu.VMEM((_SUPER * _CHUNK,), jnp.int32),
            pltpu.VMEM((_CHUNK, D), jnp.float32),
            pltpu.VMEM((_CHUNK, D), jnp.float32),
            pltpu.SemaphoreType.DMA,
            pltpu.SemaphoreType.DMA,
        ],
        compiler_params=pltpu.CompilerParams(use_tc_tiling_on_sc=False),
    )(_embed_body)

    out = embed(idx_flat, table)
    return out.reshape(R, C, D)
